# baseline (speedup 1.0000x reference)
import functools

import jax
import jax.numpy as jnp
from jax import lax
from jax.experimental import pallas as pl
from jax.experimental.pallas import tpu as pltpu
from jax.experimental.pallas import tpu_sc as plsc

_NUM_CORES = 2
_NUM_SUBCORES = 16
_NW = _NUM_CORES * _NUM_SUBCORES
_LANES = 16
_CHUNK_ROWS = 32


@functools.lru_cache(maxsize=None)
def _make_sc_hinge(B: int, C: int):
    R = B // _NW
    n_chunks = R // _CHUNK_ROWS
    chunk = _CHUNK_ROWS * C

    mesh = plsc.VectorSubcoreMesh(core_axis_name="c", subcore_axis_name="s")

    @functools.partial(
        pl.kernel,
        mesh=mesh,
        out_type=jax.ShapeDtypeStruct((B,), jnp.float32),
        compiler_params=pltpu.CompilerParams(needs_layout_passes=False),
        scratch_types=[
            pltpu.VMEM((R,), jnp.int32),
            pltpu.VMEM((R,), jnp.float32),
            pltpu.VMEM((_CHUNK_ROWS, C), jnp.float32),
            pltpu.VMEM((_CHUNK_ROWS, C), jnp.float32),
            pltpu.SemaphoreType.DMA,
            pltpu.SemaphoreType.DMA,
        ],
    )
    def sc_hinge(x_hbm, y_hbm, out_hbm, y_v, out_v, xb0, xb1, sem0, sem1):
        wid = lax.axis_index("s") * _NUM_CORES + lax.axis_index("c")
        base = wid * R
        pltpu.sync_copy(y_hbm.at[pl.ds(base, R)], y_v)

        bufs = (xb0, xb1)
        sems = (sem0, sem1)

        def start(ci):
            b = ci % 2
            return pltpu.async_copy(
                x_hbm.at[pl.ds(base + ci * _CHUNK_ROWS, _CHUNK_ROWS), :],
                bufs[b],
                sems[b],
            )

        pending = [start(0), None]
        lanes = lax.iota(jnp.int32, _LANES)
        zero = jnp.zeros((_LANES,), jnp.float32)

        for ci in range(n_chunks):
            if ci + 1 < n_chunks:
                pending[(ci + 1) % 2] = start(ci + 1)
            pending[ci % 2].wait()
            buf = bufs[ci % 2]
            for g in range(_CHUNK_ROWS // _LANES):
                loc = ci * _CHUNK_ROWS + g * _LANES
                rows = lanes + g * _LANES
                yv = y_v[pl.ds(loc, _LANES)]
                t = plsc.load_gather(buf, [rows, yv])
                a = 1.0 - t

                def jbody(i, carry, buf=buf, rows=rows, a=a):
                    a0, a1, a2, a3 = carry
                    j = i * 4
                    x0 = plsc.load_gather(buf, [rows, jnp.full((_LANES,), j, jnp.int32)])
                    x1 = plsc.load_gather(buf, [rows, jnp.full((_LANES,), j + 1, jnp.int32)])
                    x2 = plsc.load_gather(buf, [rows, jnp.full((_LANES,), j + 2, jnp.int32)])
                    x3 = plsc.load_gather(buf, [rows, jnp.full((_LANES,), j + 3, jnp.int32)])
                    a0 = a0 + jnp.maximum(x0 + a, 0.0)
                    a1 = a1 + jnp.maximum(x1 + a, 0.0)
                    a2 = a2 + jnp.maximum(x2 + a, 0.0)
                    a3 = a3 + jnp.maximum(x3 + a, 0.0)
                    return (a0, a1, a2, a3)

                a0, a1, a2, a3 = lax.fori_loop(
                    0, C // 4, jbody, (zero, zero, zero, zero)
                )
                acc = (a0 + a1) + (a2 + a3)
                out_v[pl.ds(loc, _LANES)] = (acc - 1.0) * (1.0 / C)

        pltpu.sync_copy(out_v, out_hbm.at[pl.ds(base, R)])

    return sc_hinge


@jax.jit
def kernel(output, y):
    B, C = output.shape
    y32 = y.astype(jnp.int32)
    return _make_sc_hinge(B, C)(output, y32)


# speedup vs baseline: 1.7610x; 1.7610x over previous
import functools

import jax
import jax.numpy as jnp
from jax import lax
from jax.experimental import pallas as pl
from jax.experimental.pallas import tpu as pltpu
from jax.experimental.pallas import tpu_sc as plsc

_NUM_CORES = 2
_NUM_SUBCORES = 16
_NW = _NUM_CORES * _NUM_SUBCORES
_LANES = 16
_CHUNK_ROWS = 32


@functools.lru_cache(maxsize=None)
def _make_sc_hinge(B: int, C: int):
    R = B // _NW
    n_chunks = R // _CHUNK_ROWS
    chunk = _CHUNK_ROWS * C

    n_full = C // _LANES
    tail = C - n_full * _LANES
    unroll = 8
    n_loop = n_full // unroll
    n_extra = n_full - n_loop * unroll

    mesh = plsc.VectorSubcoreMesh(core_axis_name="c", subcore_axis_name="s")

    @functools.partial(
        pl.kernel,
        mesh=mesh,
        out_type=jax.ShapeDtypeStruct((B,), jnp.float32),
        compiler_params=pltpu.CompilerParams(needs_layout_passes=False),
        scratch_types=[
            pltpu.VMEM((R,), jnp.int32),
            pltpu.VMEM((R,), jnp.float32),
            pltpu.VMEM((chunk,), jnp.float32),
            pltpu.VMEM((chunk,), jnp.float32),
            pltpu.SemaphoreType.DMA,
            pltpu.SemaphoreType.DMA,
        ],
    )
    def sc_hinge(x_hbm, y_hbm, out_hbm, y_v, out_v, xb0, xb1, sem0, sem1):
        wid = lax.axis_index("s") * _NUM_CORES + lax.axis_index("c")
        base = wid * R
        pltpu.sync_copy(y_hbm.at[pl.ds(base, R)], y_v)

        bufs = (xb0, xb1)
        sems = (sem0, sem1)

        def start(ci):
            b = ci % 2
            return pltpu.async_copy(
                x_hbm.at[pl.ds((base + ci * _CHUNK_ROWS) * C, chunk)],
                bufs[b],
                sems[b],
            )

        pending = [start(0), None]
        lanes = lax.iota(jnp.int32, _LANES)
        tail_keep = (lanes >= (_LANES - tail)) if tail else None
        zero = jnp.zeros((_LANES,), jnp.float32)

        for ci in range(n_chunks):
            if ci + 1 < n_chunks:
                pending[(ci + 1) % 2] = start(ci + 1)
            pending[ci % 2].wait()
            buf = bufs[ci % 2]

            for g in range(_CHUNK_ROWS // _LANES):
                locg = ci * _CHUNK_ROWS + g * _LANES
                yv = y_v[pl.ds(locg, _LANES)]
                t_vec = plsc.load_gather(
                    buf, [(lanes + g * _LANES) * C + yv]
                )
                a_vec = 1.0 - t_vec

                def row_fn(r2, loss_acc, buf=buf, a_vec=a_vec, g=g):
                    off = (g * _LANES + r2) * C
                    a = jnp.sum(jnp.where(lanes == r2, a_vec, 0.0))

                    def jbody(ji, carry, buf=buf, off=off, a=a):
                        accs = list(carry)
                        jo = off + ji * (unroll * _LANES)
                        for k in range(unroll):
                            x = buf[pl.ds(jo + k * _LANES, _LANES)]
                            accs[k % 4] = accs[k % 4] + jnp.maximum(
                                x + a, 0.0)
                        return tuple(accs)

                    accs = lax.fori_loop(
                        0, n_loop, jbody, (zero, zero, zero, zero)
                    )
                    accs = list(accs)
                    for k in range(n_extra):
                        x = buf[pl.ds(off + (n_loop * unroll + k) * _LANES,
                                      _LANES)]
                        accs[k % 4] = accs[k % 4] + jnp.maximum(x + a, 0.0)
                    if tail:
                        x = buf[pl.ds(off + C - _LANES, _LANES)]
                        h = jnp.maximum(x + a, 0.0)
                        accs[3] = accs[3] + jnp.where(tail_keep, h, 0.0)
                    acc = (accs[0] + accs[1]) + (accs[2] + accs[3])
                    loss = (jnp.sum(acc) - 1.0) * (1.0 / C)
                    return jnp.where(lanes == r2, loss, loss_acc)

                loss_vec = lax.fori_loop(0, _LANES, row_fn, zero)
                out_v[pl.ds(locg, _LANES)] = loss_vec

        pltpu.sync_copy(out_v, out_hbm.at[pl.ds(base, R)])

    return sc_hinge


@jax.jit
def kernel(output, y):
    B, C = output.shape
    y32 = y.astype(jnp.int32)
    return _make_sc_hinge(B, C)(output.reshape(B * C), y32)


# speedup vs baseline: 2.7706x; 1.5733x over previous
import functools

import jax
import jax.numpy as jnp
from jax import lax
from jax.experimental import pallas as pl
from jax.experimental.pallas import tpu as pltpu
from jax.experimental.pallas import tpu_sc as plsc

_NUM_CORES = 2
_NUM_SUBCORES = 16
_NW = _NUM_CORES * _NUM_SUBCORES
_LANES = 16
_CHUNK_ROWS = 32


@functools.lru_cache(maxsize=None)
def _make_sc_hinge(B: int, C: int):
    R = B // _NW
    n_chunks = R // _CHUNK_ROWS

    n_full = C // _LANES
    tail = C - n_full * _LANES
    unroll = 8
    n_loop = n_full // unroll
    n_extra = n_full - n_loop * unroll

    mesh = plsc.VectorSubcoreMesh(core_axis_name="c", subcore_axis_name="s")

    @functools.partial(
        pl.kernel,
        mesh=mesh,
        out_type=jax.ShapeDtypeStruct((B,), jnp.float32),
        compiler_params=pltpu.CompilerParams(needs_layout_passes=False),
        scratch_types=[
            pltpu.VMEM((R,), jnp.int32),
            pltpu.VMEM((R,), jnp.float32),
            pltpu.VMEM((_CHUNK_ROWS, C), jnp.float32),
            pltpu.VMEM((_CHUNK_ROWS, C), jnp.float32),
            pltpu.SemaphoreType.DMA,
            pltpu.SemaphoreType.DMA,
        ],
    )
    def sc_hinge(x_hbm, y_hbm, out_hbm, y_v, out_v, xb0, xb1, sem0, sem1):
        wid = lax.axis_index("s") * _NUM_CORES + lax.axis_index("c")
        base = wid * R
        pltpu.sync_copy(y_hbm.at[pl.ds(base, R)], y_v)

        bufs = (xb0, xb1)
        sems = (sem0, sem1)

        def start(ci):
            b = ci % 2
            return pltpu.async_copy(
                x_hbm.at[pl.ds(base + ci * _CHUNK_ROWS, _CHUNK_ROWS), :],
                bufs[b],
                sems[b],
            )

        pending = [start(0), None]
        lanes = lax.iota(jnp.int32, _LANES)
        tail_keep = (lanes >= (_LANES - tail)) if tail else None
        zero = jnp.zeros((_LANES,), jnp.float32)

        for ci in range(n_chunks):
            if ci + 1 < n_chunks:
                pending[(ci + 1) % 2] = start(ci + 1)
            pending[ci % 2].wait()
            buf = bufs[ci % 2]

            for g in range(_CHUNK_ROWS // _LANES):
                locg = ci * _CHUNK_ROWS + g * _LANES
                yv = y_v[pl.ds(locg, _LANES)]
                t_vec = plsc.load_gather(buf, [lanes + g * _LANES, yv])
                a_vec = 1.0 - t_vec

                def row_fn(r2, loss_acc, buf=buf, a_vec=a_vec, g=g):
                    row = g * _LANES + r2
                    a = jnp.sum(jnp.where(lanes == r2, a_vec, 0.0))

                    def jbody(ji, carry, buf=buf, row=row, a=a):
                        accs = list(carry)
                        jo = ji * (unroll * _LANES)
                        for k in range(unroll):
                            x = buf[row, pl.ds(jo + k * _LANES, _LANES)]
                            accs[k % 4] = accs[k % 4] + jnp.maximum(
                                x + a, 0.0)
                        return tuple(accs)

                    accs = lax.fori_loop(
                        0, n_loop, jbody, (zero, zero, zero, zero)
                    )
                    accs = list(accs)
                    for k in range(n_extra):
                        x = buf[row, pl.ds((n_loop * unroll + k) * _LANES,
                                           _LANES)]
                        accs[k % 4] = accs[k % 4] + jnp.maximum(x + a, 0.0)
                    if tail:
                        x = buf[row, pl.ds(C - _LANES, _LANES)]
                        h = jnp.maximum(x + a, 0.0)
                        accs[3] = accs[3] + jnp.where(tail_keep, h, 0.0)
                    acc = (accs[0] + accs[1]) + (accs[2] + accs[3])
                    loss = (jnp.sum(acc) - 1.0) * (1.0 / C)
                    return jnp.where(lanes == r2, loss, loss_acc)

                loss_vec = lax.fori_loop(0, _LANES, row_fn, zero)
                out_v[pl.ds(locg, _LANES)] = loss_vec

        pltpu.sync_copy(out_v, out_hbm.at[pl.ds(base, R)])

    return sc_hinge


@jax.jit
def kernel(output, y):
    B, C = output.shape
    y32 = y.astype(jnp.int32)
    return _make_sc_hinge(B, C)(output, y32)
